# baseline (speedup 1.0000x reference)
import jax
import jax.numpy as jnp
from jax import lax
from jax.experimental import pallas as pl
from jax.experimental.pallas import tpu as pltpu
from jax.experimental.pallas import tpu_sc as plsc

NC = 2
NS = 16
NW = NC * NS

BLK = 256
CHUNK = 64


def _dispatch_body(x_hbm, tok_pad_hbm, xs_hbm, i_v, rowsA_v, rowsB_v,
                   sem, isem):
    c = lax.axis_index("c")
    s = lax.axis_index("s")
    wid = s * NC + c
    pad_total = tok_pad_hbm.shape[0]
    per_w = pad_total // NW
    base = wid * per_w
    pltpu.sync_copy(tok_pad_hbm.at[pl.ds(base, per_w)], i_v)
    prev = None
    for ch in range(per_w // CHUNK):
        buf = rowsA_v if ch % 2 == 0 else rowsB_v
        g = pltpu.async_copy(
            x_hbm.at[i_v.at[pl.ds(ch * CHUNK, CHUNK)]], buf, sem)
        g.wait()
        if prev is not None:
            prev.wait()
        prev = pltpu.async_copy(
            buf, xs_hbm.at[pl.ds(base + ch * CHUNK, CHUNK)], isem)
    prev.wait()


def _combine_body(y_hbm, inv0_hbm, inv1_hbm, z0_hbm, z1_hbm,
                  i0_v, i1_v, rows0_v, rows1_v, sem, isem):
    c = lax.axis_index("c")
    s = lax.axis_index("s")
    wid = s * NC + c
    n_tokens = inv0_hbm.shape[0]
    per_w = n_tokens // NW
    base = wid * per_w
    li0 = pltpu.async_copy(inv0_hbm.at[pl.ds(base, per_w)], i0_v, isem)
    li1 = pltpu.async_copy(inv1_hbm.at[pl.ds(base, per_w)], i1_v, isem)
    li0.wait()
    li1.wait()
    g0a = pltpu.async_copy(y_hbm.at[i0_v.at[pl.ds(0, CHUNK)]], rows0_v, sem)
    g0b = pltpu.async_copy(y_hbm.at[i0_v.at[pl.ds(CHUNK, CHUNK)]], rows1_v, sem)
    g0a.wait()
    s0a = pltpu.async_copy(rows0_v, z0_hbm.at[pl.ds(base, CHUNK)], isem)
    g0b.wait()
    s0b = pltpu.async_copy(rows1_v, z0_hbm.at[pl.ds(base + CHUNK, CHUNK)], isem)
    s0a.wait()
    g1a = pltpu.async_copy(y_hbm.at[i1_v.at[pl.ds(0, CHUNK)]], rows0_v, sem)
    s0b.wait()
    g1b = pltpu.async_copy(y_hbm.at[i1_v.at[pl.ds(CHUNK, CHUNK)]], rows1_v, sem)
    g1a.wait()
    s1a = pltpu.async_copy(rows0_v, z1_hbm.at[pl.ds(base, CHUNK)], isem)
    g1b.wait()
    s1b = pltpu.async_copy(rows1_v, z1_hbm.at[pl.ds(base + CHUNK, CHUNK)], isem)
    s1a.wait()
    s1b.wait()


def _gmm_body(map_ref, xs_ref, w_ref, b_ref, g_ref, y_ref):
    e = map_ref[pl.program_id(0)]
    x = xs_ref[...].astype(jnp.bfloat16)
    y = jnp.dot(x, w_ref[e], preferred_element_type=jnp.float32)
    g = g_ref[0].reshape(-1, 1)
    y_ref[...] = (y + b_ref[e]) * g


def _pair_add_body(z0_ref, z1_ref, out_ref):
    out_ref[...] = z0_ref[...] + z1_ref[...]


def kernel(input_batch, probabilities, indices, W, b):
    n_tokens, d_model = input_batch.shape
    n_experts, _, d_out = W.shape
    top_k = indices.shape[1]
    n_slots = n_tokens * top_k
    pad_total = n_slots + n_experts * BLK
    nb = pad_total // BLK
    i32 = jnp.int32
    f32 = jnp.float32
    bf16 = jnp.bfloat16

    e_flat = indices.astype(i32).reshape(-1)
    onehot = (e_flat[:, None] == jnp.arange(n_experts, dtype=i32)).astype(i32)
    csum = jnp.cumsum(onehot, axis=0)
    counts = csum[-1]
    rank = jnp.take_along_axis(csum, e_flat[:, None], axis=1)[:, 0] - 1
    padded = ((counts + BLK - 1) // BLK) * BLK
    pstart = jnp.concatenate(
        [jnp.zeros((1,), i32), jnp.cumsum(padded)[:-1].astype(i32)])
    dest = pstart[e_flat] + rank
    dest0 = dest[0::2]
    dest1 = dest[1::2]
    block_e = jnp.clip(
        jnp.searchsorted(pstart, jnp.arange(nb, dtype=i32) * BLK,
                         side="right") - 1,
        0, n_experts - 1).astype(i32)
    tok_flat = jnp.arange(n_slots, dtype=i32) // top_k
    gate_flat = probabilities.astype(f32).reshape(-1)
    tok_pad = jnp.zeros((pad_total,), i32).at[dest].set(tok_flat)
    gpad = jnp.zeros((pad_total,), f32).at[dest].set(gate_flat)
    w_bf = W.astype(bf16)

    mesh = plsc.VectorSubcoreMesh(core_axis_name="c", subcore_axis_name="s")

    xs = pl.kernel(
        _dispatch_body,
        out_type=jax.ShapeDtypeStruct((pad_total, d_model), f32),
        mesh=mesh,
        scratch_types=[
            pltpu.VMEM((pad_total // NW,), i32),
            pltpu.VMEM((CHUNK, d_model), f32),
            pltpu.VMEM((CHUNK, d_model), f32),
            pltpu.SemaphoreType.DMA,
            pltpu.SemaphoreType.DMA,
        ],
    )(input_batch, tok_pad)

    grid_spec = pltpu.PrefetchScalarGridSpec(
        num_scalar_prefetch=1,
        grid=(nb,),
        in_specs=[
            pl.BlockSpec((BLK, d_model), lambda i, m: (i, 0)),
            pl.BlockSpec((n_experts, d_model, d_out), lambda i, m: (0, 0, 0)),
            pl.BlockSpec((n_experts, 1, d_out), lambda i, m: (0, 0, 0)),
            pl.BlockSpec((1, 1, BLK), lambda i, m: (i, 0, 0)),
        ],
        out_specs=pl.BlockSpec((BLK, d_out), lambda i, m: (i, 0)),
    )
    y_sorted = pl.pallas_call(
        _gmm_body,
        grid_spec=grid_spec,
        out_shape=jax.ShapeDtypeStruct((pad_total, d_out), f32),
    )(block_e, xs, w_bf, b.reshape(n_experts, 1, d_out),
      gpad.reshape(nb, 1, BLK))

    z0, z1 = pl.kernel(
        _combine_body,
        out_type=(jax.ShapeDtypeStruct((n_tokens, d_out), f32),
                  jax.ShapeDtypeStruct((n_tokens, d_out), f32)),
        mesh=mesh,
        scratch_types=[
            pltpu.VMEM((n_tokens // NW,), i32),
            pltpu.VMEM((n_tokens // NW,), i32),
            pltpu.VMEM((CHUNK, d_out), f32),
            pltpu.VMEM((CHUNK, d_out), f32),
            pltpu.SemaphoreType.DMA,
            pltpu.SemaphoreType.DMA,
        ],
    )(y_sorted, dest0, dest1)

    tb = 512
    out = pl.pallas_call(
        _pair_add_body,
        grid=(n_tokens // tb,),
        in_specs=[pl.BlockSpec((tb, d_out), lambda t: (t, 0)),
                  pl.BlockSpec((tb, d_out), lambda t: (t, 0))],
        out_specs=pl.BlockSpec((tb, d_out), lambda t: (t, 0)),
        out_shape=jax.ShapeDtypeStruct((n_tokens, d_out), f32),
    )(z0, z1)

    total_loss = jnp.asarray(0.0, dtype=f32)
    return (out, total_loss)


# speedup vs baseline: 4.8687x; 4.8687x over previous
import jax
import jax.numpy as jnp
from jax.experimental import pallas as pl
from jax.experimental.pallas import tpu as pltpu

TOKEN_BLOCK = 512


def _moe_dense_body(idx_ref, prob_ref, x_ref, w_ref, b_ref, out_ref):
    idx = idx_ref[...]
    p = prob_ref[...]
    x = x_ref[...].astype(jnp.bfloat16)
    n_experts = w_ref.shape[0]
    acc = None
    for e in range(n_experts):
        gate = jnp.sum(jnp.where(idx == e, p, 0.0), axis=1)
        y = jnp.dot(x, w_ref[e].astype(jnp.bfloat16),
                    preferred_element_type=jnp.float32)
        y = y + b_ref[e]
        contrib = gate[:, None] * y
        acc = contrib if acc is None else acc + contrib
    out_ref[...] = acc


def kernel(input_batch, probabilities, indices, W, b):
    n_tokens, d_model = input_batch.shape
    n_experts, _, d_out = W.shape
    idx32 = indices.astype(jnp.int32)
    grid = (n_tokens // TOKEN_BLOCK,)
    out = pl.pallas_call(
        _moe_dense_body,
        grid=grid,
        in_specs=[
            pl.BlockSpec((TOKEN_BLOCK, idx32.shape[1]), lambda t: (t, 0)),
            pl.BlockSpec((TOKEN_BLOCK, probabilities.shape[1]), lambda t: (t, 0)),
            pl.BlockSpec((TOKEN_BLOCK, d_model), lambda t: (t, 0)),
            pl.BlockSpec((n_experts, d_model, d_out), lambda t: (0, 0, 0)),
            pl.BlockSpec((n_experts, 1, d_out), lambda t: (0, 0, 0)),
        ],
        out_specs=pl.BlockSpec((TOKEN_BLOCK, d_out), lambda t: (t, 0)),
        out_shape=jax.ShapeDtypeStruct((n_tokens, d_out), input_batch.dtype),
    )(idx32, probabilities, input_batch, W, b.reshape(n_experts, 1, d_out))
    total_loss = jnp.asarray(0.0, dtype=jnp.float32)
    return (out, total_loss)
